# Initial kernel scaffold; baseline (speedup 1.0000x reference)
import jax
import jax.numpy as jnp
from jax.experimental import pallas as pl


def kernel(x, table):
    raise NotImplementedError("write your pallas kernel here")


# speedup vs baseline: 1.8759x; 1.8759x over previous
import functools

import jax
import jax.numpy as jnp
from jax import lax
from jax.experimental import pallas as pl
from jax.experimental.pallas import tpu as pltpu
from jax.experimental.pallas import tpu_sc as plsc

EMBED = 64
CHUNK = 128
NBUF = 8


@functools.cache
def _build(num_rows: int):
    mesh = plsc.VectorSubcoreMesh(core_axis_name="c", subcore_axis_name="s")
    nc, ns = mesh.num_cores, mesh.num_subcores
    nw = nc * ns
    rows_per_w = num_rows // nw
    assert rows_per_w * nw == num_rows and rows_per_w % CHUNK == 0
    n_chunks = rows_per_w // CHUNK
    steady = n_chunks - NBUF
    assert steady % NBUF == 0

    def body(idx_hbm, table_hbm, out_hbm, idx_v, rows_v, *sems):
        c = lax.axis_index("c")
        s = lax.axis_index("s")
        wid = s * nc + c
        pltpu.sync_copy(idx_hbm.at[wid], idx_v)

        def start_gather(g, b):
            pltpu.async_copy(table_hbm.at[idx_v.at[g]], rows_v.at[b], sems[b])

        def wait_gather(b):
            pltpu.make_async_copy(
                table_hbm.at[idx_v.at[0]], rows_v.at[b], sems[b]
            ).wait()

        for b in range(NBUF):
            start_gather(b, b)

        def steady_body(g0, u):
            for b in range(NBUF):
                g = g0 + b
                wait_gather(b)
                pltpu.sync_copy(rows_v.at[b], out_hbm.at[wid].at[g])
                start_gather(g + NBUF, b)
            return u

        lax.fori_loop(0, steady // NBUF,
                      lambda i, u: steady_body(i * NBUF, u), 0)

        for b in range(NBUF):
            g = steady + b
            wait_gather(b)
            pltpu.sync_copy(rows_v.at[b], out_hbm.at[wid].at[g])

    grid_kernel = pl.kernel(
        body,
        out_type=jax.ShapeDtypeStruct((nw, n_chunks, CHUNK, EMBED),
                                      jnp.float32),
        mesh=mesh,
        compiler_params=pltpu.CompilerParams(use_tc_tiling_on_sc=False),
        scratch_types=(
            [pltpu.VMEM((n_chunks, CHUNK), jnp.int32),
             pltpu.VMEM((NBUF, CHUNK, EMBED), jnp.float32)]
            + [pltpu.SemaphoreType.DMA] * NBUF
        ),
    )
    return grid_kernel, nw, n_chunks


def kernel(x, table):
    b0, b1 = x.shape
    num_rows = b0 * b1
    grid_kernel, nw, n_chunks = _build(num_rows)
    idx = x.reshape(-1).astype(jnp.int32).reshape(nw, n_chunks, CHUNK)
    out = grid_kernel(idx, table)
    return out.reshape(b0, b1, EMBED)


# speedup vs baseline: 2.0630x; 1.0997x over previous
import functools

import jax
import jax.numpy as jnp
from jax import lax
from jax.experimental import pallas as pl
from jax.experimental.pallas import tpu as pltpu
from jax.experimental.pallas import tpu_sc as plsc

EMBED = 64
CHUNK = 128
SKEW = 129
NBUF = 4


@functools.cache
def _build(n_seq: int, n_batch: int):
    mesh = plsc.VectorSubcoreMesh(core_axis_name="c", subcore_axis_name="s")
    nc, ns = mesh.num_cores, mesh.num_subcores
    nw = nc * ns
    n_bt = n_batch // CHUNK
    n_et = EMBED // 8
    n_blocks = n_seq * n_bt
    blocks_per_w = n_blocks // nw
    assert blocks_per_w * nw == n_blocks
    steady = blocks_per_w - NBUF
    assert steady % NBUF == 0

    def body(idx_hbm, table_hbm, out_hbm, idx_v, rows_v, tbuf_v, *sems):
        c = lax.axis_index("c")
        s = lax.axis_index("s")
        wid = s * nc + c
        base = wid * blocks_per_w
        pltpu.sync_copy(idx_hbm.at[pl.ds(base, blocks_per_w)], idx_v)
        iota = lax.iota(jnp.int32, 16)

        def start_gather(g, b):
            pltpu.async_copy(table_hbm.at[idx_v.at[g]], rows_v.at[b], sems[b])

        def wait_gather(b):
            pltpu.make_async_copy(
                table_hbm.at[idx_v.at[0]], rows_v.at[b], sems[b]
            ).wait()

        def emit_block(g, b):
            wait_gather(b)
            rows = rows_v.at[b]
            tb = tbuf_v.at[b]

            def trans_body(t, u):
                col = jnp.full((16,), t, jnp.int32)
                for q in range(EMBED // 16):
                    vals = rows[t, pl.ds(q * 16, 16)]
                    plsc.store_scatter(tb, [iota + q * 16, col], vals)
                return u

            lax.fori_loop(0, CHUNK, trans_body, 0)
            blk = base + g
            s_id = blk // n_bt
            bt = blk % n_bt
            for et in range(n_et):
                pltpu.sync_copy(tb.at[pl.ds(et * 8, 8), pl.ds(0, CHUNK)],
                                out_hbm.at[s_id, et, bt])

        for b in range(NBUF):
            start_gather(b, b)

        def steady_body(g0, u):
            for b in range(NBUF):
                g = g0 + b
                emit_block(g, b)
                start_gather(g + NBUF, b)
            return u

        lax.fori_loop(0, steady // NBUF,
                      lambda i, u: steady_body(i * NBUF, u), 0)

        for b in range(NBUF):
            emit_block(steady + b, b)

    grid_kernel = pl.kernel(
        body,
        out_type=jax.ShapeDtypeStruct((n_seq, n_et, n_bt, 8, CHUNK),
                                      jnp.float32),
        mesh=mesh,
        compiler_params=pltpu.CompilerParams(use_tc_tiling_on_sc=False,
                                             needs_layout_passes=False),
        scratch_types=(
            [pltpu.VMEM((blocks_per_w, CHUNK), jnp.int32),
             pltpu.VMEM((NBUF, CHUNK, EMBED), jnp.float32),
             pltpu.VMEM((NBUF, EMBED, SKEW), jnp.float32)]
            + [pltpu.SemaphoreType.DMA] * NBUF
        ),
    )
    return grid_kernel, n_bt, n_et


def kernel(x, table):
    b0, b1 = x.shape
    grid_kernel, n_bt, n_et = _build(b1, b0)
    idx = x.T.astype(jnp.int32).reshape(b1 * n_bt, CHUNK)
    out5d = grid_kernel(idx, table)
    return out5d.transpose(2, 4, 0, 1, 3).reshape(b0, b1, EMBED)


# speedup vs baseline: 2.0924x; 1.0143x over previous
import functools

import jax
import jax.numpy as jnp
from jax import lax
from jax.experimental import pallas as pl
from jax.experimental.pallas import tpu as pltpu
from jax.experimental.pallas import tpu_sc as plsc

EMBED = 64
CHUNK = 128
SKEW = 129
NBUF = 4
TUNROLL = 8


@functools.cache
def _build(n_seq: int, n_batch: int):
    mesh = plsc.VectorSubcoreMesh(core_axis_name="c", subcore_axis_name="s")
    nc, ns = mesh.num_cores, mesh.num_subcores
    nw = nc * ns
    n_bt = n_batch // CHUNK
    n_et = EMBED // 8
    n_blocks = n_seq * n_bt
    blocks_per_w = n_blocks // nw
    assert blocks_per_w * nw == n_blocks
    steady = blocks_per_w - NBUF
    assert steady % NBUF == 0

    def body(idx_hbm, table_hbm, out_hbm, idx_v, rows_v, tbuf_v, *sems):
        c = lax.axis_index("c")
        s = lax.axis_index("s")
        wid = s * nc + c
        base = wid * blocks_per_w
        pltpu.sync_copy(idx_hbm.at[pl.ds(base, blocks_per_w)], idx_v)
        iota = lax.iota(jnp.int32, 16)

        def start_gather(g, b):
            pltpu.async_copy(table_hbm.at[idx_v.at[g]], rows_v.at[b], sems[b])

        def wait_gather(b):
            pltpu.make_async_copy(
                table_hbm.at[idx_v.at[0]], rows_v.at[b], sems[b]
            ).wait()

        def emit_block(g, b):
            wait_gather(b)
            rows = rows_v.at[b]
            tb = tbuf_v.at[b]

            def trans_body(t0, u):
                for dt in range(TUNROLL):
                    t = t0 * TUNROLL + dt
                    col = jnp.full((16,), t, jnp.int32)
                    for q in range(EMBED // 16):
                        vals = rows[t, pl.ds(q * 16, 16)]
                        plsc.store_scatter(tb, [iota + q * 16, col], vals)
                return u

            lax.fori_loop(0, CHUNK // TUNROLL, trans_body, 0)
            blk = base + g
            s_id = blk // n_bt
            bt = blk % n_bt
            for et in range(n_et):
                pltpu.sync_copy(tb.at[pl.ds(et * 8, 8), pl.ds(0, CHUNK)],
                                out_hbm.at[s_id, et, bt])

        for b in range(NBUF):
            start_gather(b, b)

        def steady_body(g0, u):
            for b in range(NBUF):
                g = g0 + b
                emit_block(g, b)
                start_gather(g + NBUF, b)
            return u

        lax.fori_loop(0, steady // NBUF,
                      lambda i, u: steady_body(i * NBUF, u), 0)

        for b in range(NBUF):
            emit_block(steady + b, b)

    grid_kernel = pl.kernel(
        body,
        out_type=jax.ShapeDtypeStruct((n_seq, n_et, n_bt, 8, CHUNK),
                                      jnp.float32),
        mesh=mesh,
        compiler_params=pltpu.CompilerParams(use_tc_tiling_on_sc=False,
                                             needs_layout_passes=False),
        scratch_types=(
            [pltpu.VMEM((blocks_per_w, CHUNK), jnp.int32),
             pltpu.VMEM((NBUF, CHUNK, EMBED), jnp.float32),
             pltpu.VMEM((NBUF, EMBED, SKEW), jnp.float32)]
            + [pltpu.SemaphoreType.DMA] * NBUF
        ),
    )
    return grid_kernel, n_bt, n_et


def kernel(x, table):
    b0, b1 = x.shape
    grid_kernel, n_bt, n_et = _build(b1, b0)
    idx = x.T.astype(jnp.int32).reshape(b1 * n_bt, CHUNK)
    out5d = grid_kernel(idx, table)
    return out5d.transpose(2, 4, 0, 1, 3).reshape(b0, b1, EMBED)


# speedup vs baseline: 2.3777x; 1.1364x over previous
import functools

import jax
import jax.numpy as jnp
from jax import lax
from jax.experimental import pallas as pl
from jax.experimental.pallas import tpu as pltpu
from jax.experimental.pallas import tpu_sc as plsc

EMBED = 64
CHUNK = 128
SKEW = 129
NBUF = 4
TUNROLL = 8


@functools.cache
def _build(n_seq: int, n_batch: int):
    mesh = plsc.VectorSubcoreMesh(core_axis_name="c", subcore_axis_name="s")
    nc, ns = mesh.num_cores, mesh.num_subcores
    nw = nc * ns
    n_bt = n_batch // CHUNK
    n_et = EMBED // 8
    n_blocks = n_seq * n_bt
    blocks_per_w = n_blocks // nw
    assert blocks_per_w * nw == n_blocks
    steady = blocks_per_w - NBUF
    assert steady % NBUF == 0

    def body(idx_hbm, table_hbm, out_hbm, idx_v, rows_v, tbuf_v, *sems):
        gsems = sems[:NBUF]
        ssems = sems[NBUF:]
        c = lax.axis_index("c")
        s = lax.axis_index("s")
        wid = s * nc + c
        base = wid * blocks_per_w
        pltpu.sync_copy(idx_hbm.at[pl.ds(base, blocks_per_w)], idx_v)
        iota = lax.iota(jnp.int32, 16)
        iota_hi = iota >> 3
        iota_lo = iota & 7

        def start_gather(g, b):
            pltpu.async_copy(table_hbm.at[idx_v.at[g]], rows_v.at[b], gsems[b])

        def wait_gather(b):
            pltpu.make_async_copy(
                table_hbm.at[idx_v.at[0]], rows_v.at[b], gsems[b]
            ).wait()

        def transpose(b):
            rows = rows_v.at[b]
            tb = tbuf_v.at[b]

            def trans_body(t0, u):
                for dt in range(TUNROLL):
                    t = t0 * TUNROLL + dt
                    col = jnp.full((16,), t, jnp.int32)
                    for q in range(EMBED // 16):
                        vals = rows[t, pl.ds(q * 16, 16)]
                        plsc.store_scatter(
                            tb, [iota_hi + 2 * q, iota_lo, col], vals)
                return u

            lax.fori_loop(0, CHUNK // TUNROLL, trans_body, 0)

        def start_store(g, b):
            blk = base + g
            s_id = blk // n_bt
            bt = blk % n_bt
            pltpu.async_copy(tbuf_v.at[b, :, :, pl.ds(0, CHUNK)],
                             out_hbm.at[s_id, :, bt], ssems[b])

        def wait_store(b):
            pltpu.make_async_copy(tbuf_v.at[b, :, :, pl.ds(0, CHUNK)],
                                  out_hbm.at[0, :, 0], ssems[b]).wait()

        for b in range(NBUF):
            start_gather(b, b)
        for b in range(NBUF):
            wait_gather(b)
            transpose(b)
            start_store(b, b)
            start_gather(b + NBUF, b)

        def steady_body(g0, u):
            for b in range(NBUF):
                g = g0 + b
                wait_gather(b)
                wait_store(b)
                transpose(b)
                start_store(g, b)
                start_gather(g + NBUF, b)
            return u

        lax.fori_loop(1, steady // NBUF,
                      lambda i, u: steady_body(i * NBUF, u), 0)

        for b in range(NBUF):
            g = steady + b
            wait_gather(b)
            wait_store(b)
            transpose(b)
            start_store(g, b)
        for b in range(NBUF):
            wait_store(b)

    grid_kernel = pl.kernel(
        body,
        out_type=jax.ShapeDtypeStruct((n_seq, n_et, n_bt, 8, CHUNK),
                                      jnp.float32),
        mesh=mesh,
        compiler_params=pltpu.CompilerParams(use_tc_tiling_on_sc=False,
                                             needs_layout_passes=False),
        scratch_types=(
            [pltpu.VMEM((blocks_per_w, CHUNK), jnp.int32),
             pltpu.VMEM((NBUF, CHUNK, EMBED), jnp.float32),
             pltpu.VMEM((NBUF, n_et, 8, SKEW), jnp.float32)]
            + [pltpu.SemaphoreType.DMA] * (2 * NBUF)
        ),
    )
    return grid_kernel, n_bt, n_et


def kernel(x, table):
    b0, b1 = x.shape
    grid_kernel, n_bt, n_et = _build(b1, b0)
    idx = x.T.astype(jnp.int32).reshape(b1 * n_bt, CHUNK)
    out5d = grid_kernel(idx, table)
    return out5d.transpose(2, 4, 0, 1, 3).reshape(b0, b1, EMBED)


# speedup vs baseline: 2.9542x; 1.2425x over previous
import functools

import jax
import jax.numpy as jnp
from jax import lax
from jax.experimental import pallas as pl
from jax.experimental.pallas import tpu as pltpu
from jax.experimental.pallas import tpu_sc as plsc

EMBED = 64
CHUNK = 128
SKEW = 129
NBUF = 4
TUNROLL = 8


@functools.cache
def _build(n_seq: int, n_batch: int):
    mesh = plsc.VectorSubcoreMesh(core_axis_name="c", subcore_axis_name="s")
    nc, ns = mesh.num_cores, mesh.num_subcores
    nw = nc * ns
    n_bt = n_batch // CHUNK
    n_et = EMBED // 8
    n_blocks = n_seq * n_bt
    blocks_per_w = n_blocks // nw
    assert blocks_per_w * nw == n_blocks
    steady = blocks_per_w - NBUF
    assert steady % NBUF == 0

    def body(idx_hbm, table_hbm, out_hbm, idx_v, rows_v, tbuf_v, *sems):
        gsems = sems[:NBUF]
        ssems = sems[NBUF:]
        c = lax.axis_index("c")
        s = lax.axis_index("s")
        wid = s * nc + c
        base = wid * blocks_per_w
        pltpu.sync_copy(idx_hbm.at[pl.ds(base, blocks_per_w)], idx_v)
        iota = lax.iota(jnp.int32, 16)
        iota_hi = iota >> 3
        iota_lo = iota & 7

        def start_gather(g, b):
            pltpu.async_copy(table_hbm.at[idx_v.at[g]], rows_v.at[b], gsems[b])

        def wait_gather(b):
            pltpu.make_async_copy(
                table_hbm.at[idx_v.at[0]], rows_v.at[b], gsems[b]
            ).wait()

        def transpose(b):
            rows = rows_v.at[b]
            tb = tbuf_v.at[b]

            def trans_body(t0, u):
                for dt in range(TUNROLL):
                    t = t0 * TUNROLL + dt
                    col = jnp.full((16,), t, jnp.int32)
                    for q in range(EMBED // 16):
                        vals = rows[t, pl.ds(q * 16, 16)]
                        plsc.store_scatter(
                            tb, [iota_hi + 2 * q, iota_lo, col], vals)
                return u

            lax.fori_loop(0, CHUNK // TUNROLL, trans_body, 0)

        def start_store(g, b):
            blk = base + g
            s_id = blk // n_bt
            bt = blk % n_bt
            pltpu.async_copy(tbuf_v.at[b, :, :, pl.ds(0, CHUNK)],
                             out_hbm.at[s_id, :, bt], ssems[b])

        def wait_store(b):
            pltpu.make_async_copy(tbuf_v.at[b, :, :, pl.ds(0, CHUNK)],
                                  out_hbm.at[0, :, 0], ssems[b]).wait()

        for b in range(NBUF):
            start_gather(b, b)
        for b in range(NBUF):
            wait_gather(b)
            transpose(b)
            start_store(b, b)
            start_gather(b + NBUF, b)

        def steady_body(g0, u):
            for b in range(NBUF):
                g = g0 + b
                wait_gather(b)
                wait_store(b)
                transpose(b)
                start_store(g, b)
                start_gather(g + NBUF, b)
            return u

        lax.fori_loop(1, steady // NBUF,
                      lambda i, u: steady_body(i * NBUF, u), 0)

        for b in range(NBUF):
            g = steady + b
            wait_gather(b)
            wait_store(b)
            transpose(b)
            start_store(g, b)
        for b in range(NBUF):
            wait_store(b)

    grid_kernel = pl.kernel(
        body,
        out_type=jax.ShapeDtypeStruct((n_seq, n_et, n_bt, 8, CHUNK),
                                      jnp.float32),
        mesh=mesh,
        compiler_params=pltpu.CompilerParams(use_tc_tiling_on_sc=False,
                                             needs_layout_passes=False),
        scratch_types=(
            [pltpu.VMEM((blocks_per_w, CHUNK), jnp.int32),
             pltpu.VMEM((NBUF, CHUNK, EMBED), jnp.float32),
             pltpu.VMEM((NBUF, n_et, 8, SKEW), jnp.float32)]
            + [pltpu.SemaphoreType.DMA] * (2 * NBUF)
        ),
    )
    return grid_kernel, n_bt, n_et


TBV = 61 * 128


def _tpose_block(t_ref, o_ref):
    t = t_ref[...]
    y = jnp.transpose(t)
    y3 = y.reshape(TBV // 2, 2, EMBED)
    o_ref[...] = jnp.concatenate([y3[:, 0, :], y3[:, 1, :]], axis=1)


@functools.cache
def _build_tpose(v: int):
    return pl.pallas_call(
        _tpose_block,
        grid=((v + TBV - 1) // TBV,),
        in_specs=[pl.BlockSpec((EMBED, TBV), lambda i: (0, i))],
        out_specs=pl.BlockSpec((TBV // 2, 2 * EMBED), lambda i: (i, 0)),
        out_shape=jax.ShapeDtypeStruct((v // 2, 2 * EMBED), jnp.float32),
    )


def kernel(x, table):
    b0, b1 = x.shape
    grid_kernel, n_bt, n_et = _build(b1, b0)
    idx = x.T.astype(jnp.int32).reshape(b1 * n_bt, CHUNK)
    v, d = table.shape
    trows = _build_tpose(v)(table.T).reshape(v, d)
    out5d = grid_kernel(idx, trows)
    return out5d.transpose(2, 4, 0, 1, 3).reshape(b0, b1, EMBED)
